# Initial kernel scaffold; baseline (speedup 1.0000x reference)
import jax
import jax.numpy as jnp
from jax.experimental import pallas as pl


def kernel(x, edge_index, edge_attr, edge_type, batch, Wroot1, Wrel1, Wedge1, b1, Wroot2, Wrel2, Wedge2, b2, Wroot3, Wrel3, Wedge3, b3, Wroot4, Wrel4, Wedge4, b4, Wlin, blin):
    raise NotImplementedError("write your pallas kernel here")


# speedup vs baseline: 15.4771x; 15.4771x over previous
import jax
import jax.numpy as jnp
from jax import lax
from jax.experimental import pallas as pl
from jax.experimental.pallas import tpu as pltpu
from jax.experimental.pallas import tpu_sc as plsc

N_NODES = 10000
N_EDGES = 320000
DIM = 128
N_REL = 2
N_GRAPH = 16

NC = 2
NS = 16
CHUNK = 128
NCH = 80
NCH0 = 120
NCH1 = 40
TOTAL_CH = NCH0 + NCH1
E_PAD = NS * TOTAL_CH * CHUNK
N_PAD = 10112
ROWS_PER_TILE = N_PAD // NS

_SC_MESH = plsc.VectorSubcoreMesh(core_axis_name="c", subcore_axis_name="s",
                                  num_cores=NC, num_subcores=NS)


DST_BITS = 14


def _sc_edges_body(xr_hbm, pidx_hbm, part_hbm,
                   pidx_v, gidx_c, dst_c, rows_v, sem0, sem1, accum):
    c = lax.axis_index("c")
    s = lax.axis_index("s")

    def zrow(i, _):
        for j in range(DIM // 16):
            rows_v[0, i, pl.ds(j * 16, 16)] = jnp.zeros((16,), jnp.float32)
        return 0
    lax.fori_loop(0, CHUNK, zrow, 0)
    base = s * ROWS_PER_TILE
    for k in range(ROWS_PER_TILE // CHUNK):
        pltpu.sync_copy(rows_v.at[0], accum.at[pl.ds(base + k * CHUNK, CHUNK)])
    rem = ROWS_PER_TILE % CHUNK
    if rem:
        pltpu.sync_copy(
            rows_v.at[0, pl.ds(0, rem)],
            accum.at[pl.ds(base + (ROWS_PER_TILE // CHUNK) * CHUNK, rem)])
    plsc.subcore_barrier()

    def unpack(j, slot):
        for k in range(CHUNK // 16):
            w = pidx_v[j, pl.ds(k * 16, 16)]
            gidx_c[slot, pl.ds(k * 16, 16)] = jnp.right_shift(w, DST_BITS)
            dst_c[slot, pl.ds(k * 16, 16)] = jnp.bitwise_and(
                w, jnp.int32((1 << DST_BITS) - 1))

    def gather(slot, sem):
        pltpu.async_copy(xr_hbm.at[gidx_c.at[slot]], rows_v.at[slot], sem)

    def scatter(slot):
        pltpu.sync_copy(rows_v.at[slot], accum.at[dst_c.at[slot]], add=True)

    def run_span(start, nch):
        pltpu.sync_copy(pidx_hbm.at[s, pl.ds(start, nch)],
                        pidx_v.at[pl.ds(0, nch)])

        unpack(0, 0)
        gather(0, sem0)

        def step(j, _):
            @pl.when(j % 2 == 1)
            def _():
                unpack(j, 1)
                gather(1, sem1)
                pltpu.make_async_copy(xr_hbm.at[gidx_c.at[0]], rows_v.at[0],
                                      sem0).wait()
                scatter(0)

            @pl.when(j % 2 == 0)
            def _():
                unpack(j, 0)
                gather(0, sem0)
                pltpu.make_async_copy(xr_hbm.at[gidx_c.at[1]], rows_v.at[1],
                                      sem1).wait()
                scatter(1)
            return 0
        lax.fori_loop(1, nch, step, 0)
        pltpu.make_async_copy(xr_hbm.at[gidx_c.at[1]], rows_v.at[1],
                              sem1).wait()
        scatter(1)

    @pl.when(c == 0)
    def _():
        run_span(0, NCH0)

    @pl.when(c == 1)
    def _():
        run_span(NCH0, NCH1)

    plsc.subcore_barrier()
    pltpu.sync_copy(accum.at[pl.ds(base, ROWS_PER_TILE)],
                    part_hbm.at[c, pl.ds(base, ROWS_PER_TILE)])


_sc_edges = pl.kernel(
    _sc_edges_body,
    out_type=jax.ShapeDtypeStruct((NC, N_PAD, DIM), jnp.float32),
    mesh=_SC_MESH,
    scratch_types=[
        pltpu.VMEM((NCH0, CHUNK), jnp.int32),
        pltpu.VMEM((2, CHUNK), jnp.int32),
        pltpu.VMEM((2, CHUNK), jnp.int32),
        pltpu.VMEM((2, CHUNK, DIM), jnp.float32),
        pltpu.SemaphoreType.DMA,
        pltpu.SemaphoreType.DMA,
        pltpu.VMEM_SHARED((N_PAD, DIM), jnp.float32),
    ],
    compiler_params=pltpu.CompilerParams(needs_layout_passes=False),
)


def _sc_stats_body(eattr_hbm, dst_hbm, deg_hbm, se_hbm,
                   eattr_v, dst_v, deg_l, se_l, tmp16):
    c = lax.axis_index("c")
    s = lax.axis_index("s")

    zf = jnp.zeros((16,), jnp.float32)

    def zstep(i, _):
        deg_l[pl.ds(i * 16, 16)] = zf
        se_l[pl.ds(i * 16, 16)] = zf
        return 0
    lax.fori_loop(0, N_PAD // 16, zstep, 0)

    pltpu.sync_copy(eattr_hbm.at[c, s], eattr_v)
    pltpu.sync_copy(dst_hbm.at[c, s], dst_v)

    lanes = lax.iota(jnp.int32, 16)

    def vec_step(t, _):
        j = t // (CHUNK // 16)
        k = t % (CHUNK // 16)
        dvec = dst_v[j, pl.ds(k * 16, 16)]
        evec = eattr_v[j, pl.ds(k * 16, 16)]
        key, val = plsc.sort_key_val(dvec, evec)
        prev = key.at[jnp.maximum(lanes - 1, 0)].get(mode="promise_in_bounds")
        first = (lanes == 0) | (key != prev)
        nxt = key.at[jnp.minimum(lanes + 1, 15)].get(mode="promise_in_bounds")
        last = (lanes == 15) | (key != nxt)
        segid = plsc.cumsum(first.astype(jnp.int32))
        plsc.store_scatter(tmp16, [segid - 1], lanes, mask=first)
        start = plsc.load_gather(tmp16, [segid - 1])
        csum = plsc.cumsum(val)
        pb = csum.at[jnp.maximum(start - 1, 0)].get(mode="promise_in_bounds")
        pb = jnp.where(start == 0, 0.0, pb)
        seg_sum = csum - pb
        seg_cnt = (lanes - start + 1).astype(jnp.float32)
        plsc.addupdate_scatter(deg_l, [key], seg_cnt, mask=last)
        plsc.addupdate_scatter(se_l, [key], seg_sum, mask=last)
        return 0
    lax.fori_loop(0, NCH * (CHUNK // 16), vec_step, 0)

    pltpu.sync_copy(deg_l, deg_hbm.at[c, s])
    pltpu.sync_copy(se_l, se_hbm.at[c, s])


_sc_stats = pl.kernel(
    _sc_stats_body,
    out_type=(jax.ShapeDtypeStruct((NC, NS, N_PAD), jnp.float32),
              jax.ShapeDtypeStruct((NC, NS, N_PAD), jnp.float32)),
    mesh=_SC_MESH,
    scratch_types=[
        pltpu.VMEM((NCH, CHUNK), jnp.float32),
        pltpu.VMEM((NCH, CHUNK), jnp.int32),
        pltpu.VMEM((N_PAD,), jnp.float32),
        pltpu.VMEM((N_PAD,), jnp.float32),
        pltpu.VMEM((16,), jnp.int32),
    ],
    compiler_params=pltpu.CompilerParams(needs_layout_passes=False),
)


BN = 1264


def _prep_body(h_ref, wrel_ref, wroot_ref, xr_ref, hroot_ref):
    h = h_ref[...]
    xr_ref[0] = jnp.dot(h, wrel_ref[0], preferred_element_type=jnp.float32)
    xr_ref[1] = jnp.dot(h, wrel_ref[1], preferred_element_type=jnp.float32)
    hroot_ref[...] = jnp.dot(h, wroot_ref[...],
                             preferred_element_type=jnp.float32)


def _tc_prep(h, wrel, wroot):
    grid = N_PAD // BN
    return pl.pallas_call(
        _prep_body,
        grid=(grid,),
        in_specs=[
            pl.BlockSpec((BN, DIM), lambda i: (i, 0)),
            pl.BlockSpec((N_REL, DIM, DIM), lambda i: (0, 0, 0)),
            pl.BlockSpec((DIM, DIM), lambda i: (0, 0)),
        ],
        out_specs=[
            pl.BlockSpec((N_REL, BN, DIM), lambda i: (0, i, 0)),
            pl.BlockSpec((BN, DIM), lambda i: (i, 0)),
        ],
        out_shape=[
            jax.ShapeDtypeStruct((N_REL, N_PAD, DIM), jnp.float32),
            jax.ShapeDtypeStruct((N_PAD, DIM), jnp.float32),
        ],
    )(h, wrel, wroot)


def _nstats_body(degp_ref, sep_ref, dinv_ref, ses_ref):
    for i in range(N_PAD // BN):
        deg = jnp.sum(degp_ref[:, pl.ds(i * BN, BN)], axis=0)
        dinv_ref[i, :] = 1.0 / jnp.maximum(deg, 1.0)
        ses_ref[i, :] = jnp.sum(sep_ref[:, pl.ds(i * BN, BN)], axis=0)


def _tc_nstats(degp, sep):
    return pl.pallas_call(
        _nstats_body,
        grid=(1,),
        in_specs=[
            pl.BlockSpec((NC * NS, N_PAD), lambda i: (0, 0)),
            pl.BlockSpec((NC * NS, N_PAD), lambda i: (0, 0)),
        ],
        out_specs=[
            pl.BlockSpec((N_PAD // BN, BN), lambda i: (0, 0)),
            pl.BlockSpec((N_PAD // BN, BN), lambda i: (0, 0)),
        ],
        out_shape=[
            jax.ShapeDtypeStruct((N_PAD // BN, BN), jnp.float32),
            jax.ShapeDtypeStruct((N_PAD // BN, BN), jnp.float32),
        ],
    )(degp, sep)


def _elu_update(part_ref, hroot_ref, dinv_ref, ses_ref, wedge_ref, b_ref):
    i = pl.program_id(0)
    dinv = dinv_ref[i][:, None]
    se = ses_ref[i][:, None]
    agg = part_ref[0] + part_ref[1] + se * wedge_ref[...]
    z = hroot_ref[...] + agg * dinv + b_ref[...]
    return jnp.where(z > 0, z, jnp.exp(jnp.minimum(z, 0.0)) - 1.0)


def _combine_body(part_ref, hroot_ref, dinv_ref, ses_ref, wedge_ref, b_ref,
                  h_ref):
    h_ref[...] = _elu_update(part_ref, hroot_ref, dinv_ref, ses_ref,
                             wedge_ref, b_ref)


def _fused_body(part_ref, hroot_ref, dinv_ref, ses_ref, wedge_ref, b_ref,
                wrel_ref, wroot_ref, xr_ref, hroot_out_ref):
    h = _elu_update(part_ref, hroot_ref, dinv_ref, ses_ref, wedge_ref, b_ref)
    xr_ref[0] = jnp.dot(h, wrel_ref[0], preferred_element_type=jnp.float32)
    xr_ref[1] = jnp.dot(h, wrel_ref[1], preferred_element_type=jnp.float32)
    hroot_out_ref[...] = jnp.dot(h, wroot_ref[...],
                                 preferred_element_type=jnp.float32)


def _tc_fused(part, hroot, dinv, ses, wedge, b, wrel, wroot):
    grid = N_PAD // BN
    return pl.pallas_call(
        _fused_body,
        grid=(grid,),
        in_specs=[
            pl.BlockSpec((NC, BN, DIM), lambda i: (0, i, 0)),
            pl.BlockSpec((BN, DIM), lambda i: (i, 0)),
            pl.BlockSpec((N_PAD // BN, BN), lambda i: (0, 0)),
            pl.BlockSpec((N_PAD // BN, BN), lambda i: (0, 0)),
            pl.BlockSpec((1, DIM), lambda i: (0, 0)),
            pl.BlockSpec((1, DIM), lambda i: (0, 0)),
            pl.BlockSpec((N_REL, DIM, DIM), lambda i: (0, 0, 0)),
            pl.BlockSpec((DIM, DIM), lambda i: (0, 0)),
        ],
        out_specs=[
            pl.BlockSpec((N_REL, BN, DIM), lambda i: (0, i, 0)),
            pl.BlockSpec((BN, DIM), lambda i: (i, 0)),
        ],
        out_shape=[
            jax.ShapeDtypeStruct((N_REL, N_PAD, DIM), jnp.float32),
            jax.ShapeDtypeStruct((N_PAD, DIM), jnp.float32),
        ],
    )(part, hroot, dinv, ses, wedge, b, wrel, wroot)


def _tc_combine(part, hroot, dinv, ses, wedge, b):
    grid = N_PAD // BN
    return pl.pallas_call(
        _combine_body,
        grid=(grid,),
        in_specs=[
            pl.BlockSpec((NC, BN, DIM), lambda i: (0, i, 0)),
            pl.BlockSpec((BN, DIM), lambda i: (i, 0)),
            pl.BlockSpec((N_PAD // BN, BN), lambda i: (0, 0)),
            pl.BlockSpec((N_PAD // BN, BN), lambda i: (0, 0)),
            pl.BlockSpec((1, DIM), lambda i: (0, 0)),
            pl.BlockSpec((1, DIM), lambda i: (0, 0)),
        ],
        out_specs=pl.BlockSpec((BN, DIM), lambda i: (i, 0)),
        out_shape=jax.ShapeDtypeStruct((N_PAD, DIM), jnp.float32),
    )(part, hroot, dinv, ses, wedge, b)


def _pool_body(h_ref, oh_ref, wlin_ref, blin_ref, out_ref, acc, cacc):
    i = pl.program_id(0)

    @pl.when(i == 0)
    def _():
        acc[...] = jnp.zeros_like(acc)
        cacc[...] = jnp.zeros_like(cacc)

    oh = oh_ref[...]
    hb = h_ref[...]
    acc[...] += jnp.dot(oh.T, hb, preferred_element_type=jnp.float32)
    cacc[...] += jnp.dot(oh.T, jnp.ones_like(hb),
                         preferred_element_type=jnp.float32)

    @pl.when(i == pl.num_programs(0) - 1)
    def _():
        pooled = acc[...] / jnp.maximum(cacc[...], 1.0)
        out_ref[...] = jnp.dot(pooled, wlin_ref[...],
                               preferred_element_type=jnp.float32) + blin_ref[...]


def _tc_pool(h, onehot, wlin_pad, blin_pad):
    grid = N_PAD // BN
    return pl.pallas_call(
        _pool_body,
        grid=(grid,),
        in_specs=[
            pl.BlockSpec((BN, DIM), lambda i: (i, 0)),
            pl.BlockSpec((BN, N_GRAPH), lambda i: (i, 0)),
            pl.BlockSpec((DIM, DIM), lambda i: (0, 0)),
            pl.BlockSpec((1, DIM), lambda i: (0, 0)),
        ],
        out_specs=pl.BlockSpec((N_GRAPH, DIM), lambda i: (0, 0)),
        out_shape=jax.ShapeDtypeStruct((N_GRAPH, DIM), jnp.float32),
        scratch_shapes=[
            pltpu.VMEM((N_GRAPH, DIM), jnp.float32),
            pltpu.VMEM((N_GRAPH, DIM), jnp.float32),
        ],
    )(h, onehot, wlin_pad, blin_pad)


def kernel(x, edge_index, edge_attr, edge_type, batch,
           Wroot1, Wrel1, Wedge1, b1,
           Wroot2, Wrel2, Wedge2, b2,
           Wroot3, Wrel3, Wedge3, b3,
           Wroot4, Wrel4, Wedge4, b4,
           Wlin, blin):
    x = jnp.pad(x.astype(jnp.float32), ((0, N_PAD - N_NODES), (0, 0)))
    src = edge_index[0].astype(jnp.int32)
    dst = edge_index[1].astype(jnp.int32)
    etype = edge_type.astype(jnp.int32)

    pad = E_PAD - N_EDGES
    gidx = etype * N_PAD + src
    pidx = jnp.pad((gidx << DST_BITS) | dst, (0, pad),
                   constant_values=N_NODES)
    pidx = pidx.reshape(NS, TOTAL_CH, CHUNK)
    dstp = jnp.pad(dst, (0, pad), constant_values=N_NODES)
    dstp = dstp.reshape(NC, NS, NCH, CHUNK)
    eattrp = jnp.pad(edge_attr[:, 0].astype(jnp.float32), (0, pad))
    eattrp = eattrp.reshape(NC, NS, NCH, CHUNK)

    degp, sep = _sc_stats(eattrp, dstp)
    dinv, ses = _tc_nstats(degp.reshape(NC * NS, N_PAD),
                           sep.reshape(NC * NS, N_PAD))

    layers = [(Wroot1, Wrel1, Wedge1, b1), (Wroot2, Wrel2, Wedge2, b2),
              (Wroot3, Wrel3, Wedge3, b3), (Wroot4, Wrel4, Wedge4, b4)]

    xr, hroot = _tc_prep(x, layers[0][1], layers[0][0])
    for li in range(4):
        wedge, bb = layers[li][2].reshape(1, DIM), layers[li][3].reshape(1, DIM)
        part = _sc_edges(xr.reshape(N_REL * N_PAD, DIM), pidx)
        if li < 3:
            xr, hroot = _tc_fused(part, hroot, dinv, ses, wedge, bb,
                                  layers[li + 1][1], layers[li + 1][0])
        else:
            h = _tc_combine(part, hroot, dinv, ses, wedge, bb)

    batchp = jnp.pad(batch.astype(jnp.int32), (0, N_PAD - N_NODES),
                     constant_values=N_GRAPH)
    onehot = (batchp[:, None] == jnp.arange(N_GRAPH)[None, :]).astype(jnp.float32)
    wlin_pad = jnp.zeros((DIM, DIM), jnp.float32).at[:, :Wlin.shape[1]].set(Wlin)
    blin_pad = jnp.zeros((1, DIM), jnp.float32).at[0, :blin.shape[0]].set(blin)
    out = _tc_pool(h, onehot, wlin_pad, blin_pad)
    return out[:, :blin.shape[0]]
